# baseline (speedup 1.0000x reference)
import functools
import math

import jax
import jax.numpy as jnp
from jax import lax
from jax.experimental import pallas as pl
from jax.experimental.pallas import tpu as pltpu
from jax.experimental.pallas import tpu_sc as plsc

D_MODEL = 64
SCALE = math.sqrt(D_MODEL)
NUM_WORKERS = 32
X_ROWS = 4096
X_COLS = 200
T_PAIRS = X_COLS // 2
B_SLAB = X_ROWS // NUM_WORKERS
NRING = 2
LANES = 16

BB = 512


def _make_sc_gather():
    mesh = plsc.VectorSubcoreMesh(core_axis_name="c", subcore_axis_name="s")

    @functools.partial(
        pl.kernel,
        out_type=jax.ShapeDtypeStruct((T_PAIRS, X_ROWS, 2 * D_MODEL),
                                      jnp.float32),
        mesh=mesh,
        compiler_params=pltpu.CompilerParams(
            use_tc_tiling_on_sc=False,
            needs_layout_passes=False,
        ),
        scratch_types=(
            [pltpu.VMEM((X_COLS, B_SLAB), jnp.int32)]
            + [pltpu.VMEM((T_PAIRS, 2 * B_SLAB), jnp.int32)]
            + [pltpu.VMEM((2 * B_SLAB, D_MODEL), jnp.float32)] * NRING
            + [pltpu.VMEM((B_SLAB, 2 * D_MODEL), jnp.float32)] * NRING
            + [pltpu.SemaphoreType.DMA] * (2 * NRING)
        ),
    )
    def gather_scale(xt_hbm, table_hbm, mid_hbm, xt_v, xt_p, *bufs_and_sems):
        rows = list(bufs_and_sems[:NRING])
        obuf = list(bufs_and_sems[NRING:2 * NRING])
        gsem = list(bufs_and_sems[2 * NRING:3 * NRING])
        osem = list(bufs_and_sems[3 * NRING:])
        wid = lax.axis_index("s") * 2 + lax.axis_index("c")
        col0 = wid * B_SLAB

        pltpu.sync_copy(xt_hbm.at[:, pl.ds(col0, B_SLAB)], xt_v)

        @plsc.parallel_loop(0, T_PAIRS, unroll=2)
        def repack_body(p):
            for e in range(2):
                for j in range(B_SLAB // LANES):
                    src = pl.ds(j * LANES, LANES)
                    dst = pl.ds(e * B_SLAB + j * LANES, LANES)
                    xt_p[p, dst] = xt_v[p * 2 + e, src]

        def gather_desc(p, r):
            src = table_hbm.at[xt_p.at[p]]
            return pltpu.make_async_copy(src, rows[r], gsem[r])

        def out_desc(p, r):
            dst = mid_hbm.at[p, pl.ds(col0, B_SLAB), :]
            return pltpu.make_async_copy(obuf[r], dst, osem[r])

        gather_desc(0, 0).start()

        def pair_body(h, carry):
            for r in range(NRING):
                p = h * NRING + r
                gather_desc(p, r).wait()

                @pl.when(p >= NRING)
                def _wait_prev_out():
                    out_desc(p - NRING, r).wait()

                @pl.when(p + 1 < T_PAIRS)
                def _start_next_gather():
                    gather_desc(p + 1, (r + 1) % NRING).start()

                @plsc.parallel_loop(0, B_SLAB, unroll=2)
                def merge_body(q):
                    for e in range(2):
                        for j in range(D_MODEL // LANES):
                            src = pl.ds(j * LANES, LANES)
                            dst = pl.ds(e * D_MODEL + j * LANES, LANES)
                            obuf[r][q, dst] = (
                                rows[r][e * B_SLAB + q, src] * SCALE
                            )

                out_desc(p, r).start()
            return carry

        lax.fori_loop(0, T_PAIRS // NRING, pair_body, 0)
        for r in range(NRING):
            out_desc(T_PAIRS - NRING + r, r).wait()

    return gather_scale


def _tc_body(m_ref, o_ref):
    m = m_ref[0]
    for e in range(2):
        o_ref[e] = jnp.transpose(m[:, e * D_MODEL:(e + 1) * D_MODEL], (1, 0))


def _make_tc_transpose():
    return pl.pallas_call(
        _tc_body,
        grid=(T_PAIRS, X_ROWS // BB),
        in_specs=[pl.BlockSpec((1, BB, 2 * D_MODEL), lambda p, c: (p, c, 0))],
        out_specs=pl.BlockSpec((2, D_MODEL, BB), lambda p, c: (p, 0, c)),
        out_shape=jax.ShapeDtypeStruct((X_COLS, D_MODEL, X_ROWS), jnp.float32),
        compiler_params=pltpu.CompilerParams(
            dimension_semantics=("arbitrary", "arbitrary"),
        ),
    )


_sc_gather = _make_sc_gather()
_tc_transpose = _make_tc_transpose()


def kernel(x, embed):
    mid = _sc_gather(x.T, embed)
    out = _tc_transpose(mid)
    return out.transpose((2, 0, 1))


# speedup vs baseline: 1.0778x; 1.0778x over previous
import functools
import math

import jax
import jax.numpy as jnp
from jax import lax
from jax.experimental import pallas as pl
from jax.experimental.pallas import tpu as pltpu
from jax.experimental.pallas import tpu_sc as plsc

D_MODEL = 64
SCALE = math.sqrt(D_MODEL)
NUM_WORKERS = 32
X_ROWS = 4096
X_COLS = 200
XR_PER_WORKER = X_ROWS // NUM_WORKERS
NBUF = 4
LANES = 16


def _make_kernel():
    mesh = plsc.VectorSubcoreMesh(core_axis_name="c", subcore_axis_name="s")

    @functools.partial(
        pl.kernel,
        out_type=jax.ShapeDtypeStruct((X_ROWS, X_COLS, D_MODEL), jnp.float32),
        mesh=mesh,
        compiler_params=pltpu.CompilerParams(use_tc_tiling_on_sc=False),
        scratch_types=(
            [pltpu.VMEM((XR_PER_WORKER, X_COLS), jnp.int32)]
            + [pltpu.VMEM((X_COLS, D_MODEL), jnp.float32)] * NBUF
            + [pltpu.SemaphoreType.DMA] * (2 * NBUF)
        ),
    )
    def gather_scale(idx_hbm, table_hbm, out_hbm, idx_all, *bufs_and_sems):
        rows = list(bufs_and_sems[:NBUF])
        gsem = list(bufs_and_sems[NBUF:2 * NBUF])
        osem = list(bufs_and_sems[2 * NBUF:])
        wid = lax.axis_index("s") * 2 + lax.axis_index("c")
        base = wid * XR_PER_WORKER

        pltpu.sync_copy(idx_hbm.at[pl.ds(base, XR_PER_WORKER)], idx_all)

        def gather_desc(c, b):
            src = table_hbm.at[idx_all.at[c]]
            return pltpu.make_async_copy(src, rows[b], gsem[b])

        def out_desc(c, b):
            return pltpu.make_async_copy(rows[b], out_hbm.at[base + c], osem[b])

        for c0 in range(NBUF - 1):
            gather_desc(c0, c0).start()

        def ring_body(p, carry):
            for b in range(NBUF):
                c = p * NBUF + b
                bprev = (b - 1) % NBUF
                gather_desc(c, b).wait()

                @pl.when(c >= 1)
                def _wait_prev_out():
                    out_desc(c - 1, bprev).wait()

                @pl.when(c + NBUF - 1 < XR_PER_WORKER)
                def _start_next_gather():
                    gather_desc(c + NBUF - 1, bprev).start()

                def scale_body(t, carry2):
                    for j in range(D_MODEL // LANES):
                        sl = pl.ds(j * LANES, LANES)
                        rows[b][t, sl] = rows[b][t, sl] * SCALE
                    return carry2

                lax.fori_loop(0, X_COLS, scale_body, 0, unroll=4)
                out_desc(c, b).start()
            return carry

        lax.fori_loop(0, XR_PER_WORKER // NBUF, ring_body, 0)
        out_desc(XR_PER_WORKER - 1, (XR_PER_WORKER - 1) % NBUF).wait()

    return gather_scale


_gather_scale = _make_kernel()


def kernel(x, embed):
    return _gather_scale(x, embed)
